# baseline (speedup 1.0000x reference)
import functools

import jax
import jax.numpy as jnp
from jax.experimental import pallas as pl
from jax.experimental.pallas import tpu as pltpu

TOP_K = 8
KEEP_RATE = 0.9
NEG_INF = float("-inf")
LANES = 128


def _kc_kernel(x_ref, w_ref, kn_ref, out_ref, aux_ref, psum_ref, dsum_ref,
               *, n_steps, n_tokens):
    i = pl.program_id(0)
    xb = x_ref[...]
    n = kn_ref.shape[0]
    t = xb.shape[0]

    logits_t = jax.lax.dot_general(
        w_ref[...], xb, (((0,), (1,)), ((), ())),
        preferred_element_type=jnp.float32)
    act_t = jax.lax.dot_general(
        kn_ref[...], xb, (((1,), (1,)), ((), ())),
        preferred_element_type=jnp.float32)

    iota = jax.lax.broadcasted_iota(jnp.int32, (n, t), 0)
    cur = logits_t
    gnum = jnp.zeros_like(logits_t)
    picked = jnp.zeros_like(logits_t, dtype=jnp.bool_)
    denom = None
    m0 = None
    for k in range(TOP_K):
        mk = jnp.max(cur, axis=0, keepdims=True)
        is_max = cur == mk
        cand = jnp.where(is_max, iota, n)
        amin = jnp.min(cand, axis=0, keepdims=True)
        sel = iota == amin
        if k == 0:
            m0 = mk
            ek = jnp.ones_like(mk)
        else:
            ek = jnp.exp(mk - m0)
        gnum = gnum + ek * sel.astype(jnp.float32)
        denom = ek if denom is None else denom + ek
        picked = picked | sel
        cur = jnp.where(sel, NEG_INF, cur)

    w_dense_t = act_t * gnum / denom
    out_ref[...] = jax.lax.dot_general(
        w_dense_t, kn_ref[...], (((0,), (0,)), ((), ())),
        preferred_element_type=jnp.float32) * (1.0 / KEEP_RATE)

    e_all = jnp.exp(logits_t - m0)
    probs_t = e_all / jnp.sum(e_all, axis=0, keepdims=True)

    @pl.when(i == 0)
    def _init():
        psum_ref[...] = jnp.zeros_like(psum_ref)
        dsum_ref[...] = jnp.zeros_like(dsum_ref)

    pickedf = picked.astype(jnp.float32)
    p_acc = psum_ref[...]
    d_acc = dsum_ref[...]
    for c in range(t // LANES):
        sl = slice(c * LANES, (c + 1) * LANES)
        p_acc = p_acc + probs_t[:, sl]
        d_acc = d_acc + pickedf[:, sl]
    psum_ref[...] = p_acc
    dsum_ref[...] = d_acc

    @pl.when(i == n_steps - 1)
    def _finalize():
        ps = jnp.sum(psum_ref[...], axis=1, keepdims=True)
        ds = jnp.sum(dsum_ref[...], axis=1, keepdims=True)
        scale = float(n) / (float(n_tokens) * float(n_tokens) * TOP_K)
        aux_ref[...] = scale * jnp.sum(ps * ds, keepdims=True)


@functools.partial(jax.jit, static_argnames=())
def _run(x, know_neurons, W_r):
    B, S, D = x.shape
    N = know_neurons.shape[0]
    tokens = B * S
    T = 2048
    n_steps = tokens // T
    xf = x.reshape(tokens, D)

    out, aux = pl.pallas_call(
        functools.partial(_kc_kernel, n_steps=n_steps, n_tokens=tokens),
        grid=(n_steps,),
        in_specs=[
            pl.BlockSpec((T, D), lambda i: (i, 0)),
            pl.BlockSpec((D, N), lambda i: (0, 0)),
            pl.BlockSpec((N, D), lambda i: (0, 0)),
        ],
        out_specs=[
            pl.BlockSpec((T, D), lambda i: (i, 0)),
            pl.BlockSpec((1, 1), lambda i: (0, 0)),
        ],
        out_shape=[
            jax.ShapeDtypeStruct((tokens, D), jnp.float32),
            jax.ShapeDtypeStruct((1, 1), jnp.float32),
        ],
        scratch_shapes=[
            pltpu.VMEM((N, LANES), jnp.float32),
            pltpu.VMEM((N, LANES), jnp.float32),
        ],
    )(xf, W_r, know_neurons)
    return out.reshape(B, S, D), aux[0, 0]


def kernel(x, know_neurons, W_r, attention_mask, deterministic):
    return _run(x, know_neurons, W_r)


# speedup vs baseline: 1.0461x; 1.0461x over previous
import functools

import jax
import jax.numpy as jnp
from jax.experimental import pallas as pl
from jax.experimental.pallas import tpu as pltpu

TOP_K = 8
KEEP_RATE = 0.9
NEG_INF = float("-inf")
LANES = 128


def _kc_kernel(x_ref, w_ref, kn_ref, out_ref, aux_ref, psum_ref, dsum_ref,
               *, n_steps, n_tokens):
    i = pl.program_id(0)
    xb = x_ref[...]
    n = kn_ref.shape[0]
    t = xb.shape[0]

    logits_t = jax.lax.dot_general(
        w_ref[...], xb, (((0,), (1,)), ((), ())),
        preferred_element_type=jnp.float32)
    act_t = jax.lax.dot_general(
        kn_ref[...], xb, (((1,), (1,)), ((), ())),
        preferred_element_type=jnp.float32)

    iota = jax.lax.broadcasted_iota(jnp.int32, (n, t), 0)
    cur = logits_t
    gnum = jnp.zeros_like(logits_t)
    picked = jnp.zeros_like(logits_t, dtype=jnp.bool_)
    denom = None
    m0 = None
    for k in range(TOP_K):
        mk = jnp.max(cur, axis=0, keepdims=True)
        is_max = cur == mk
        cand = jnp.where(is_max, iota, n)
        amin = jnp.min(cand, axis=0, keepdims=True)
        sel = iota == amin
        if k == 0:
            m0 = mk
            ek = jnp.ones_like(mk)
        else:
            ek = jnp.exp(mk - m0)
        gnum = gnum + ek * sel.astype(jnp.float32)
        denom = ek if denom is None else denom + ek
        picked = picked | sel
        cur = jnp.where(sel, NEG_INF, cur)

    w_dense_t = act_t * gnum / denom
    out_ref[...] = jax.lax.dot_general(
        w_dense_t, kn_ref[...], (((0,), (0,)), ((), ())),
        preferred_element_type=jnp.float32) * (1.0 / KEEP_RATE)

    e_all = jnp.exp(logits_t - m0)
    probs_t = e_all / jnp.sum(e_all, axis=0, keepdims=True)

    @pl.when(i == 0)
    def _init():
        psum_ref[...] = jnp.zeros_like(psum_ref)
        dsum_ref[...] = jnp.zeros_like(dsum_ref)

    pickedf = picked.astype(jnp.float32)
    p_acc = psum_ref[...]
    d_acc = dsum_ref[...]
    for c in range(t // LANES):
        sl = slice(c * LANES, (c + 1) * LANES)
        p_acc = p_acc + probs_t[:, sl]
        d_acc = d_acc + pickedf[:, sl]
    psum_ref[...] = p_acc
    dsum_ref[...] = d_acc

    @pl.when(i == n_steps - 1)
    def _finalize():
        ps = jnp.sum(psum_ref[...], axis=1, keepdims=True)
        ds = jnp.sum(dsum_ref[...], axis=1, keepdims=True)
        scale = float(n) / (float(n_tokens) * float(n_tokens) * TOP_K)
        aux_ref[...] = scale * jnp.sum(ps * ds, keepdims=True)


@functools.partial(jax.jit, static_argnames=())
def _run(x, know_neurons, W_r):
    B, S, D = x.shape
    N = know_neurons.shape[0]
    tokens = B * S
    T = 1024
    n_steps = tokens // T
    xf = x.reshape(tokens, D)

    out, aux = pl.pallas_call(
        functools.partial(_kc_kernel, n_steps=n_steps, n_tokens=tokens),
        grid=(n_steps,),
        in_specs=[
            pl.BlockSpec((T, D), lambda i: (i, 0)),
            pl.BlockSpec((D, N), lambda i: (0, 0)),
            pl.BlockSpec((N, D), lambda i: (0, 0)),
        ],
        out_specs=[
            pl.BlockSpec((T, D), lambda i: (i, 0)),
            pl.BlockSpec((1, 1), lambda i: (0, 0)),
        ],
        out_shape=[
            jax.ShapeDtypeStruct((tokens, D), jnp.float32),
            jax.ShapeDtypeStruct((1, 1), jnp.float32),
        ],
        scratch_shapes=[
            pltpu.VMEM((N, LANES), jnp.float32),
            pltpu.VMEM((N, LANES), jnp.float32),
        ],
    )(xf, W_r, know_neurons)
    return out.reshape(B, S, D), aux[0, 0]


def kernel(x, know_neurons, W_r, attention_mask, deterministic):
    return _run(x, know_neurons, W_r)
